# baseline (speedup 1.0000x reference)
import functools

import jax
import jax.numpy as jnp
from jax import lax
from jax.experimental import pallas as pl
from jax.experimental.pallas import tpu as pltpu
from jax.experimental.pallas import tpu_sc as plsc

_NUM_MASHUP = 100000
_NUM_API = 1000
_BATCH = 4096

_L = 16
_NW = 32
_B_PER_W = _BATCH // _NW
_FULL_CHUNKS = _NUM_API // _L - 1


def _sigmoid16(x):
    return 1.0 / (1.0 + jnp.exp(-x))


def _sc_body(idx_hbm, table_hbm, out_hbm, idx_v, rows_v, sem):
    wid = lax.axis_index("s") * 2 + lax.axis_index("c")
    base = wid * _B_PER_W
    pltpu.sync_copy(idx_hbm.at[pl.ds(base, _B_PER_W)], idx_v)
    pltpu.async_copy(table_hbm.at[idx_v], rows_v, sem).wait()

    def row_body(r, carry):
        def chunk_body(c, carry2):
            off = c * _L
            rows_v[r, pl.ds(off, _L)] = _sigmoid16(rows_v[r, pl.ds(off, _L)])
            return carry2

        lax.fori_loop(0, _FULL_CHUNKS, chunk_body, 0, unroll=True)
        xa = rows_v[r, pl.ds(976, _L)]
        xb = rows_v[r, pl.ds(984, _L)]
        rows_v[r, pl.ds(976, _L)] = _sigmoid16(xa)
        rows_v[r, pl.ds(984, _L)] = _sigmoid16(xb)
        return carry

    lax.fori_loop(0, _B_PER_W, row_body, 0)
    pltpu.sync_copy(rows_v, out_hbm.at[pl.ds(base, _B_PER_W)])


@jax.jit
def _pop_sc(u, m2a_mat):
    mesh = plsc.VectorSubcoreMesh(core_axis_name="c", subcore_axis_name="s")
    kfn = functools.partial(
        pl.kernel,
        mesh=mesh,
        compiler_params=pltpu.CompilerParams(use_tc_tiling_on_sc=False),
        out_type=jax.ShapeDtypeStruct((_BATCH, _NUM_API), jnp.float32),
        scratch_types=[
            pltpu.VMEM((_B_PER_W,), jnp.int32),
            pltpu.VMEM((_B_PER_W, _NUM_API), jnp.float32),
            pltpu.SemaphoreType.DMA,
        ],
    )(_sc_body)
    return kfn(u, m2a_mat)


def kernel(u, m2a_mat):
    return _pop_sc(u, m2a_mat)


# speedup vs baseline: 3.6860x; 3.6860x over previous
import functools

import jax
import jax.numpy as jnp
from jax import lax
from jax.experimental import pallas as pl
from jax.experimental.pallas import tpu as pltpu
from jax.experimental.pallas import tpu_sc as plsc

_NUM_MASHUP = 100000
_NUM_API = 1000
_BATCH = 4096

_L = 16
_NW = 32
_B_PER_W = _BATCH // _NW
_G = 8
_NGRP = _B_PER_W // _G
_FULL = _NUM_API // _L
_TAIL = _NUM_API - _L


def _sigmoid16(x):
    return 1.0 / (1.0 + jnp.exp(-x))


def _sc_body(idx_hbm, table_hbm, out_hbm, idx_v, blocks, oslab, gsem, osem):
    wid = lax.axis_index("s") * 2 + lax.axis_index("c")
    base = pl.multiple_of(wid * _B_PER_W, 8)
    pltpu.sync_copy(idx_hbm.at[pl.ds(base, _B_PER_W)],
                    idx_v.at[pl.ds(0, _B_PER_W)])

    lane = lax.iota(jnp.int32, _L)

    def group(g, carry):
        x = idx_v[pl.ds(pl.multiple_of(g * _G, 8), _L)]

        subs = []
        for jj in range(_G):
            u_j = jnp.sum(jnp.where(lane == jj, x, 0))
            blk8 = pl.multiple_of(
                lax.shift_left(lax.shift_right_logical(u_j, 3), 3), 8)
            subs.append(lax.bitwise_and(u_j, 7))
            pltpu.async_copy(
                table_hbm.at[pl.ds(blk8, 8)], blocks.at[jj], gsem)

        for jj in range(_G):
            pltpu.make_async_copy(
                table_hbm.at[pl.ds(0, 8)], blocks.at[jj], gsem).wait()

        for jj in range(_G):
            s = subs[jj]

            def chunk(c, carry2):
                off = c * _L
                oslab[jj, pl.ds(off, _L)] = _sigmoid16(
                    blocks[jj, s, pl.ds(off, _L)])
                return carry2

            lax.fori_loop(0, _FULL, chunk, 0)
            oslab[jj, pl.ds(_TAIL, _L)] = _sigmoid16(
                blocks[jj, s, pl.ds(_TAIL, _L)])

        off = pl.multiple_of(base + g * _G, 8)
        pltpu.async_copy(oslab, out_hbm.at[pl.ds(off, _G)], osem).wait()
        return carry

    lax.fori_loop(0, _NGRP, group, 0)


@jax.jit
def _pop_sc(u, m2a_mat):
    mesh = plsc.VectorSubcoreMesh(core_axis_name="c", subcore_axis_name="s")
    kfn = functools.partial(
        pl.kernel,
        mesh=mesh,
        compiler_params=pltpu.CompilerParams(needs_layout_passes=False),
        out_type=jax.ShapeDtypeStruct((_BATCH, _NUM_API), jnp.float32),
        scratch_types=[
            pltpu.VMEM((_B_PER_W + _L,), jnp.int32),
            pltpu.VMEM((_G, 8, _NUM_API), jnp.float32),
            pltpu.VMEM((_G, _NUM_API), jnp.float32),
            pltpu.SemaphoreType.DMA,
            pltpu.SemaphoreType.DMA,
        ],
    )(_sc_body)
    return kfn(u, m2a_mat)


def kernel(u, m2a_mat):
    return _pop_sc(u, m2a_mat)


# speedup vs baseline: 4.9095x; 1.3319x over previous
import functools

import jax
import jax.numpy as jnp
from jax import lax
from jax.experimental import pallas as pl
from jax.experimental.pallas import tpu as pltpu
from jax.experimental.pallas import tpu_sc as plsc

_NUM_MASHUP = 100000
_NUM_API = 1000
_BATCH = 4096

_L = 16
_NW = 32
_B_PER_W = _BATCH // _NW
_G = 4
_NPAIR = _B_PER_W // (2 * _G)
_FULL = _NUM_API // _L
_TAIL = _NUM_API - _L


def _sigmoid16(x):
    return 1.0 / (1.0 + jnp.exp(-x))


def _sc_body(idx_hbm, table_hbm, out_hbm,
             idx_v, buf_a, buf_b, oslab, gsem_a, gsem_b, osem):
    wid = lax.axis_index("s") * 2 + lax.axis_index("c")
    base = pl.multiple_of(wid * _B_PER_W, 8)
    pltpu.sync_copy(idx_hbm.at[pl.ds(base, _B_PER_W)],
                    idx_v.at[pl.ds(0, _B_PER_W)])

    lane = lax.iota(jnp.int32, _L)

    def fire4(x, lane_off, buf, sem):
        subs = []
        for jj in range(_G):
            u_j = jnp.sum(jnp.where(lane == lane_off + jj, x, 0))
            blk8 = pl.multiple_of(
                lax.shift_left(lax.shift_right_logical(u_j, 3), 3), 8)
            subs.append(lax.bitwise_and(u_j, 7))
            pltpu.async_copy(table_hbm.at[pl.ds(blk8, 8)], buf.at[jj], sem)
        return subs

    def drain4(buf, sem):
        for jj in range(_G):
            pltpu.make_async_copy(
                table_hbm.at[pl.ds(0, 8)], buf.at[jj], sem).wait()

    def compute4(buf, subs, orow0):
        for jj in range(_G):
            s = subs[jj]

            def chunk(c):
                off = c * _L
                oslab[orow0 + jj, pl.ds(off, _L)] = _sigmoid16(
                    buf[jj, s, pl.ds(off, _L)])

            plsc.parallel_loop(0, _FULL, unroll=4)(chunk)
            oslab[orow0 + jj, pl.ds(_TAIL, _L)] = _sigmoid16(
                buf[jj, s, pl.ds(_TAIL, _L)])

    x0 = idx_v[pl.ds(0, _L)]
    subs_a0 = fire4(x0, 0, buf_a, gsem_a)

    def pair(i, subs_a):
        x = idx_v[pl.ds(pl.multiple_of(i * 8, 8), _L)]
        subs_b = fire4(x, _G, buf_b, gsem_b)

        @pl.when(i > 0)
        def _():
            pltpu.make_async_copy(
                oslab, out_hbm.at[pl.ds(0, 2 * _G)], osem).wait()

        drain4(buf_a, gsem_a)
        compute4(buf_a, subs_a, 0)

        xn = idx_v[pl.ds(pl.multiple_of(i * 8 + 8, 8), _L)]
        subs_an = []
        for jj in range(_G):
            u_j = jnp.sum(jnp.where(lane == jj, xn, 0))
            subs_an.append(lax.bitwise_and(u_j, 7))

        @pl.when(i < _NPAIR - 1)
        def _():
            for jj in range(_G):
                u_j = jnp.sum(jnp.where(lane == jj, xn, 0))
                blk8 = pl.multiple_of(
                    lax.shift_left(lax.shift_right_logical(u_j, 3), 3), 8)
                pltpu.async_copy(
                    table_hbm.at[pl.ds(blk8, 8)], buf_a.at[jj], gsem_a)

        drain4(buf_b, gsem_b)
        compute4(buf_b, subs_b, _G)

        off = pl.multiple_of(base + i * 8, 8)
        pltpu.async_copy(oslab, out_hbm.at[pl.ds(off, 2 * _G)], osem)
        return tuple(subs_an)

    subs_final = lax.fori_loop(0, _NPAIR, pair, tuple(subs_a0))
    del subs_final
    pltpu.make_async_copy(oslab, out_hbm.at[pl.ds(0, 2 * _G)], osem).wait()


@jax.jit
def _pop_sc(u, m2a_mat):
    mesh = plsc.VectorSubcoreMesh(core_axis_name="c", subcore_axis_name="s")
    kfn = functools.partial(
        pl.kernel,
        mesh=mesh,
        compiler_params=pltpu.CompilerParams(needs_layout_passes=False),
        out_type=jax.ShapeDtypeStruct((_BATCH, _NUM_API), jnp.float32),
        scratch_types=[
            pltpu.VMEM((_B_PER_W + _L,), jnp.int32),
            pltpu.VMEM((_G, 8, _NUM_API), jnp.float32),
            pltpu.VMEM((_G, 8, _NUM_API), jnp.float32),
            pltpu.VMEM((2 * _G, _NUM_API), jnp.float32),
            pltpu.SemaphoreType.DMA,
            pltpu.SemaphoreType.DMA,
            pltpu.SemaphoreType.DMA,
        ],
    )(_sc_body)
    return kfn(u, m2a_mat)


def kernel(u, m2a_mat):
    return _pop_sc(u, m2a_mat)


# speedup vs baseline: 4.9477x; 1.0078x over previous
import functools

import jax
import jax.numpy as jnp
from jax import lax
from jax.experimental import pallas as pl
from jax.experimental.pallas import tpu as pltpu
from jax.experimental.pallas import tpu_sc as plsc

_NUM_MASHUP = 100000
_NUM_API = 1000
_BATCH = 4096

_L = 16
_NW = 32
_B_PER_W = _BATCH // _NW
_G = 4
_NPAIR = _B_PER_W // (2 * _G)
_FULL = _NUM_API // _L
_TAIL = _NUM_API - _L


def _sigmoid16(x):
    return 1.0 / (1.0 + jnp.exp(-x))


def _sc_body(idx_hbm, table_hbm, out_hbm,
             idx_v, buf_a, buf_b, oslab, gsem_a, gsem_b, osem):
    wid = lax.axis_index("s") * 2 + lax.axis_index("c")
    base = pl.multiple_of(wid * _B_PER_W, 8)
    pltpu.sync_copy(idx_hbm.at[pl.ds(base, _B_PER_W)],
                    idx_v.at[pl.ds(0, _B_PER_W)])

    lane = lax.iota(jnp.int32, _L)

    def fire4(x, lane_off, buf, sem):
        subs = []
        for jj in range(_G):
            u_j = jnp.sum(jnp.where(lane == lane_off + jj, x, 0))
            blk8 = pl.multiple_of(
                lax.shift_left(lax.shift_right_logical(u_j, 3), 3), 8)
            subs.append(lax.bitwise_and(u_j, 7))
            pltpu.async_copy(table_hbm.at[pl.ds(blk8, 8)], buf.at[jj], sem)
        return subs

    def drain4(buf, sem):
        for jj in range(_G):
            pltpu.make_async_copy(
                table_hbm.at[pl.ds(0, 8)], buf.at[jj], sem).wait()

    def compute4(buf, subs, orow0):
        for jj in range(_G):
            s = subs[jj]

            def chunk(c):
                off = c * _L
                oslab[orow0 + jj, pl.ds(off, _L)] = _sigmoid16(
                    buf[jj, s, pl.ds(off, _L)])

            plsc.parallel_loop(0, _FULL, unroll=8)(chunk)
            oslab[orow0 + jj, pl.ds(_TAIL, _L)] = _sigmoid16(
                buf[jj, s, pl.ds(_TAIL, _L)])

    x0 = idx_v[pl.ds(0, _L)]
    subs_a0 = fire4(x0, 0, buf_a, gsem_a)

    def pair(i, subs_a):
        x = idx_v[pl.ds(pl.multiple_of(i * 8, 8), _L)]
        subs_b = fire4(x, _G, buf_b, gsem_b)

        @pl.when(i > 0)
        def _():
            pltpu.make_async_copy(
                oslab, out_hbm.at[pl.ds(0, 2 * _G)], osem).wait()

        drain4(buf_a, gsem_a)
        compute4(buf_a, subs_a, 0)

        xn = idx_v[pl.ds(pl.multiple_of(i * 8 + 8, 8), _L)]
        subs_an = []
        for jj in range(_G):
            u_j = jnp.sum(jnp.where(lane == jj, xn, 0))
            subs_an.append(lax.bitwise_and(u_j, 7))

        @pl.when(i < _NPAIR - 1)
        def _():
            for jj in range(_G):
                u_j = jnp.sum(jnp.where(lane == jj, xn, 0))
                blk8 = pl.multiple_of(
                    lax.shift_left(lax.shift_right_logical(u_j, 3), 3), 8)
                pltpu.async_copy(
                    table_hbm.at[pl.ds(blk8, 8)], buf_a.at[jj], gsem_a)

        drain4(buf_b, gsem_b)
        compute4(buf_b, subs_b, _G)

        off = pl.multiple_of(base + i * 8, 8)
        pltpu.async_copy(oslab, out_hbm.at[pl.ds(off, 2 * _G)], osem)
        return tuple(subs_an)

    subs_final = lax.fori_loop(0, _NPAIR, pair, tuple(subs_a0))
    del subs_final
    pltpu.make_async_copy(oslab, out_hbm.at[pl.ds(0, 2 * _G)], osem).wait()


@jax.jit
def _pop_sc(u, m2a_mat):
    mesh = plsc.VectorSubcoreMesh(core_axis_name="c", subcore_axis_name="s")
    kfn = functools.partial(
        pl.kernel,
        mesh=mesh,
        compiler_params=pltpu.CompilerParams(needs_layout_passes=False),
        out_type=jax.ShapeDtypeStruct((_BATCH, _NUM_API), jnp.float32),
        scratch_types=[
            pltpu.VMEM((_B_PER_W + _L,), jnp.int32),
            pltpu.VMEM((_G, 8, _NUM_API), jnp.float32),
            pltpu.VMEM((_G, 8, _NUM_API), jnp.float32),
            pltpu.VMEM((2 * _G, _NUM_API), jnp.float32),
            pltpu.SemaphoreType.DMA,
            pltpu.SemaphoreType.DMA,
            pltpu.SemaphoreType.DMA,
        ],
    )(_sc_body)
    return kfn(u, m2a_mat)


def kernel(u, m2a_mat):
    return _pop_sc(u, m2a_mat)
